# Initial kernel scaffold; baseline (speedup 1.0000x reference)
import functools

import jax
import jax.numpy as jnp
from jax import lax
from jax.experimental import pallas as pl
from jax.experimental.pallas import tpu as pltpu
from jax.experimental.pallas import tpu_sc as plsc

N = 10000
D = 128
E = 320000
NC = 2
NS = 16
C = 128
EPT = -(-E // (NS * C)) * C
CH = EPT // C
RPT = N // NS


def _seq_fts(h, Wst):
    RA = 2000
    nb = N // RA

    def body(h_ref, w_ref, o_ref):
        o_ref[0] = jnp.dot(h_ref[...], w_ref[0],
                           preferred_element_type=jnp.float32)

    return pl.pallas_call(
        body,
        grid=(NC, nb),
        in_specs=[
            pl.BlockSpec((RA, D), lambda p, b: (b, 0)),
            pl.BlockSpec((1, D, D), lambda p, b: (p, 0, 0)),
        ],
        out_specs=pl.BlockSpec((1, RA, D), lambda p, b: (p, b, 0)),
        out_shape=jax.ShapeDtypeStruct((NC, N, D), jnp.float32),
    )(h, Wst)


def _sc_agg(seq2n, srcs, dsts, ews, zeros):
    mesh = plsc.VectorSubcoreMesh(core_axis_name="c", subcore_axis_name="s")

    @functools.partial(
        pl.kernel,
        out_type=jax.ShapeDtypeStruct((NC, N, D), jnp.float32),
        mesh=mesh,
        scratch_types=[
            pltpu.VMEM((CH, C), jnp.int32),
            pltpu.VMEM((CH, C), jnp.int32),
            pltpu.VMEM((CH, C), jnp.float32),
            pltpu.VMEM((C, D), jnp.float32),
            pltpu.VMEM_SHARED((N, D), jnp.float32),
            pltpu.SemaphoreType.DMA,
        ],
    )
    def body(seq_hbm, src_hbm, dst_hbm, ew_hbm, z_hbm, out_hbm,
             src_v, dst_v, ew_v, rows_v, agg_sh, sem):
        c = lax.axis_index("c")
        s = lax.axis_index("s")
        pltpu.sync_copy(src_hbm.at[c, s], src_v)
        pltpu.sync_copy(dst_hbm.at[c, s], dst_v)
        pltpu.sync_copy(ew_hbm.at[c, s], ew_v)
        pltpu.sync_copy(z_hbm, agg_sh.at[pl.ds(s * RPT, RPT)])
        plsc.subcore_barrier()

        def chunk(j, carry):
            pltpu.async_copy(seq_hbm.at[src_v.at[j]], rows_v, sem).wait()

            def edge(i, carry2):
                w = plsc.load_gather(
                    ew_v,
                    [jnp.full((16,), j, jnp.int32),
                     jnp.full((16,), i, jnp.int32)])
                for k in range(D // 16):
                    sl = pl.ds(k * 16, 16)
                    rows_v[i, sl] = rows_v[i, sl] * w
                return carry2

            lax.fori_loop(0, C, edge, 0)
            pltpu.sync_copy(rows_v, agg_sh.at[dst_v.at[j]], add=True)
            return carry

        lax.fori_loop(0, CH, chunk, 0)
        plsc.subcore_barrier()
        pltpu.sync_copy(agg_sh.at[pl.ds(s * RPT, RPT)],
                        out_hbm.at[c, pl.ds(s * RPT, RPT)])

    return body(seq2n, srcs, dsts, ews, zeros)


def _film_att(agg, seq, h, ntf, gt, ad, aW1, ab1, aW2, avec):
    R = 500
    nb = N // R

    def body(agg_ref, seq_ref, h_ref, nt_ref, gt_ref, ad_ref,
             aw1_ref, ab1_ref, aw2_ref, a_ref, o_ref):
        nt = nt_ref[...]

        def z(p):
            g = gt_ref[p, 0:1, :] + nt * (gt_ref[p, 1:2, :] - gt_ref[p, 0:1, :])
            b = ad_ref[p, 0:1, :] + nt * (ad_ref[p, 1:2, :] - ad_ref[p, 0:1, :])
            zz = g * agg_ref[p] + b + seq_ref[p]
            return jnp.where(zz >= 0, zz, a_ref[p] * zz)

        z0 = z(0)
        z1 = z(1)
        aw1 = aw1_ref[...]
        ab1 = ab1_ref[...]
        aw2 = aw2_ref[...]
        w0 = jnp.dot(jnp.tanh(jnp.dot(z0, aw1,
                                      preferred_element_type=jnp.float32)
                              + ab1), aw2,
                     preferred_element_type=jnp.float32)
        w1 = jnp.dot(jnp.tanh(jnp.dot(z1, aw1,
                                      preferred_element_type=jnp.float32)
                              + ab1), aw2,
                     preferred_element_type=jnp.float32)
        m = jnp.maximum(w0, w1)
        e0 = jnp.exp(w0 - m)
        e1 = jnp.exp(w1 - m)
        o_ref[...] = (e0 * z0 + e1 * z1) / (e0 + e1) + h_ref[...]

    full3 = lambda i: (0, 0, 0)
    return pl.pallas_call(
        body,
        grid=(nb,),
        in_specs=[
            pl.BlockSpec((NC, R, D), lambda i: (0, i, 0)),
            pl.BlockSpec((NC, R, D), lambda i: (0, i, 0)),
            pl.BlockSpec((R, D), lambda i: (i, 0)),
            pl.BlockSpec((R, 1), lambda i: (i, 0)),
            pl.BlockSpec((NC, 2, D), full3),
            pl.BlockSpec((NC, 2, D), full3),
            pl.BlockSpec((D, D), lambda i: (0, 0)),
            pl.BlockSpec((1, D), lambda i: (0, 0)),
            pl.BlockSpec((D, 1), lambda i: (0, 0)),
            pl.BlockSpec(memory_space=pltpu.SMEM),
        ],
        out_specs=pl.BlockSpec((R, D), lambda i: (i, 0)),
        out_shape=jax.ShapeDtypeStruct((N, D), jnp.float32),
    )(agg, seq, h, ntf, gt, ad, aW1, ab1, aW2, avec)


def _prep_edges(ei, ew, table_off):
    pad = NS * EPT - E
    src = jnp.concatenate([ei[1] + table_off,
                           jnp.zeros((pad,), jnp.int32)]).reshape(NS, CH, C)
    dst = jnp.concatenate([ei[0],
                           jnp.zeros((pad,), jnp.int32)]).reshape(NS, CH, C)
    eww = jnp.concatenate([ew,
                           jnp.zeros((pad,), jnp.float32)]).reshape(NS, CH, C)
    return src, dst, eww


def kernel(h, edge_index0, edge_index1, edge_weight0, edge_weight1,
           node_type, W0, a0, Wg0, bg0, Wb0, bb0, bias0,
           W1, a1, Wg1, bg1, Wb1, bb1, bias1, attW1, attb1, attW2):
    Wst = jnp.stack([W0, W1])
    seq = _seq_fts(h, Wst)

    s0, d0, w0 = _prep_edges(edge_index0, edge_weight0, 0)
    s1, d1, w1 = _prep_edges(edge_index1, edge_weight1, N)
    srcs = jnp.stack([s0, s1])
    dsts = jnp.stack([d0, d1])
    ews = jnp.stack([w0, w1])
    zeros = jnp.zeros((RPT, D), jnp.float32)

    agg = _sc_agg(seq.reshape(NC * N, D), srcs, dsts, ews, zeros)

    gt = jnp.stack([Wg0 + bg0[None, :], Wg1 + bg1[None, :]])
    ad = jnp.stack([Wb0 + (bb0 + bias0)[None, :],
                    Wb1 + (bb1 + bias1)[None, :]])
    avec = jnp.stack([a0, a1])
    ntf = node_type.astype(jnp.float32)[:, None]

    return _film_att(agg, seq, h, ntf, gt, ad,
                     attW1, attb1[None, :], attW2, avec)


# speedup vs baseline: 4.5303x; 4.5303x over previous
import functools

import jax
import jax.numpy as jnp
from jax import lax
from jax.experimental import pallas as pl
from jax.experimental.pallas import tpu as pltpu
from jax.experimental.pallas import tpu_sc as plsc

N = 10000
D = 128
E = 320000
NC = 2
NS = 16
C = 128
EPT = -(-E // (NS * C)) * C
CH = EPT // C
NP = 10240
RPT = NP // NS


def _seq_fts(h, Wst):
    RA = 2000
    nb = N // RA

    def body(h_ref, w_ref, o_ref):
        o_ref[0] = jnp.dot(h_ref[...], w_ref[0],
                           preferred_element_type=jnp.float32)

    return pl.pallas_call(
        body,
        grid=(NC, nb),
        in_specs=[
            pl.BlockSpec((RA, D), lambda p, b: (b, 0)),
            pl.BlockSpec((1, D, D), lambda p, b: (p, 0, 0)),
        ],
        out_specs=pl.BlockSpec((1, RA, D), lambda p, b: (p, b, 0)),
        out_shape=jax.ShapeDtypeStruct((NC, N, D), jnp.float32),
    )(h, Wst)


def _sc_agg(seq2n, ed, ews, zeros):
    mesh = plsc.VectorSubcoreMesh(core_axis_name="c", subcore_axis_name="s")

    @functools.partial(
        pl.kernel,
        out_type=jax.ShapeDtypeStruct((NC, NP, D), jnp.float32),
        mesh=mesh,
        scratch_types=[
            pltpu.VMEM((2, C), jnp.int32),
            pltpu.VMEM((1, C), jnp.float32),
            pltpu.VMEM((C, D), jnp.float32),
            pltpu.VMEM_SHARED((NP, D), jnp.float32),
            pltpu.SemaphoreType.DMA,
        ],
    )
    def body(seq_hbm, ed_hbm, ew_hbm, z_hbm, out_hbm,
             ec_v, ew_v, rows_v, agg_sh, sem):
        c = lax.axis_index("c")
        s = lax.axis_index("s")
        pltpu.sync_copy(z_hbm, agg_sh.at[pl.ds(s * RPT, RPT)])
        plsc.subcore_barrier()

        def chunk(j, carry):
            pltpu.sync_copy(ed_hbm.at[c, s, j], ec_v)
            pltpu.sync_copy(ew_hbm.at[c, s, j], ew_v)
            pltpu.async_copy(seq_hbm.at[ec_v.at[0]], rows_v, sem).wait()

            def edge_group(g, carry2):
                ewv = ew_v[0, pl.ds(g * 16, 16)]
                for r in range(16):
                    i = g * 16 + r
                    w = jnp.full((16,), ewv[r], jnp.float32)
                    for k in range(D // 16):
                        sl = pl.ds(k * 16, 16)
                        rows_v[i, sl] = rows_v[i, sl] * w
                return carry2

            lax.fori_loop(0, C // 16, edge_group, 0)
            pltpu.sync_copy(rows_v, agg_sh.at[ec_v.at[1]], add=True)
            return carry

        lax.fori_loop(0, CH, chunk, 0)
        plsc.subcore_barrier()
        pltpu.sync_copy(agg_sh.at[pl.ds(s * RPT, RPT)],
                        out_hbm.at[c, pl.ds(s * RPT, RPT)])

    return body(seq2n, ed, ews, zeros)


def _film_att(agg, seq, h, ntf, gt, ad, aW1, ab1, aW2, avec):
    R = 1000
    nb = N // R

    def body(agg_ref, seq_ref, h_ref, nt_ref, gt_ref, ad_ref,
             aw1_ref, ab1_ref, aw2_ref, a_ref, o_ref):
        nt = nt_ref[...]

        def z(p):
            g = gt_ref[p, 0:1, :] + nt * (gt_ref[p, 1:2, :] - gt_ref[p, 0:1, :])
            b = ad_ref[p, 0:1, :] + nt * (ad_ref[p, 1:2, :] - ad_ref[p, 0:1, :])
            zz = g * agg_ref[p] + b + seq_ref[p]
            return jnp.where(zz >= 0, zz, a_ref[p] * zz)

        z0 = z(0)
        z1 = z(1)
        aw1 = aw1_ref[...]
        ab1 = ab1_ref[...]
        aw2 = aw2_ref[...]
        w0 = jnp.dot(jnp.tanh(jnp.dot(z0, aw1,
                                      preferred_element_type=jnp.float32)
                              + ab1), aw2,
                     preferred_element_type=jnp.float32)
        w1 = jnp.dot(jnp.tanh(jnp.dot(z1, aw1,
                                      preferred_element_type=jnp.float32)
                              + ab1), aw2,
                     preferred_element_type=jnp.float32)
        m = jnp.maximum(w0, w1)
        e0 = jnp.exp(w0 - m)
        e1 = jnp.exp(w1 - m)
        o_ref[...] = (e0 * z0 + e1 * z1) / (e0 + e1) + h_ref[...]

    full3 = lambda i: (0, 0, 0)
    return pl.pallas_call(
        body,
        grid=(nb,),
        in_specs=[
            pl.BlockSpec((NC, R, D), lambda i: (0, i, 0)),
            pl.BlockSpec((NC, R, D), lambda i: (0, i, 0)),
            pl.BlockSpec((R, D), lambda i: (i, 0)),
            pl.BlockSpec((R, 1), lambda i: (i, 0)),
            pl.BlockSpec((NC, 2, D), full3),
            pl.BlockSpec((NC, 2, D), full3),
            pl.BlockSpec((D, D), lambda i: (0, 0)),
            pl.BlockSpec((1, D), lambda i: (0, 0)),
            pl.BlockSpec((D, 1), lambda i: (0, 0)),
            pl.BlockSpec(memory_space=pltpu.SMEM),
        ],
        out_specs=pl.BlockSpec((R, D), lambda i: (i, 0)),
        out_shape=jax.ShapeDtypeStruct((N, D), jnp.float32),
    )(agg, seq, h, ntf, gt, ad, aW1, ab1, aW2, avec)


def _prep_edges(ei, ew, table_off):
    pad = NS * EPT - E
    src = jnp.concatenate([ei[1] + table_off,
                           jnp.zeros((pad,), jnp.int32)]).reshape(NS, CH, C)
    dst = jnp.concatenate([ei[0],
                           jnp.zeros((pad,), jnp.int32)]).reshape(NS, CH, C)
    eww = jnp.concatenate([ew,
                           jnp.zeros((pad,), jnp.float32)]).reshape(NS, CH, 1, C)
    return jnp.stack([src, dst], axis=2), eww


def kernel(h, edge_index0, edge_index1, edge_weight0, edge_weight1,
           node_type, W0, a0, Wg0, bg0, Wb0, bb0, bias0,
           W1, a1, Wg1, bg1, Wb1, bb1, bias1, attW1, attb1, attW2):
    Wst = jnp.stack([W0, W1])
    seq = _seq_fts(h, Wst)

    ed0, ew0 = _prep_edges(edge_index0, edge_weight0, 0)
    ed1, ew1 = _prep_edges(edge_index1, edge_weight1, N)
    ed = jnp.stack([ed0, ed1])
    ews = jnp.stack([ew0, ew1])
    zeros = jnp.zeros((RPT, D), jnp.float32)

    agg = _sc_agg(seq.reshape(NC * N, D), ed, ews, zeros)

    gt = jnp.stack([Wg0 + bg0[None, :], Wg1 + bg1[None, :]])
    ad = jnp.stack([Wb0 + (bb0 + bias0)[None, :],
                    Wb1 + (bb1 + bias1)[None, :]])
    avec = jnp.stack([a0, a1])
    ntf = node_type.astype(jnp.float32)[:, None]

    return _film_att(agg, seq, h, ntf, gt, ad,
                     attW1, attb1[None, :], attW2, avec)


# speedup vs baseline: 5.2543x; 1.1598x over previous
import functools

import jax
import jax.numpy as jnp
from jax import lax
from jax.experimental import pallas as pl
from jax.experimental.pallas import tpu as pltpu
from jax.experimental.pallas import tpu_sc as plsc

N = 10000
D = 128
E = 320000
NC = 2
NS = 16
C = 128
CH = -(-E // (NS * C))
CH += CH % 2
EPT = CH * C
NP = 10240
RPT = NP // NS


def _seq_fts(h, Wst):
    RA = 2000
    nb = N // RA

    def body(h_ref, w_ref, o_ref):
        o_ref[0] = jnp.dot(h_ref[...], w_ref[0],
                           preferred_element_type=jnp.float32)

    return pl.pallas_call(
        body,
        grid=(NC, nb),
        in_specs=[
            pl.BlockSpec((RA, D), lambda p, b: (b, 0)),
            pl.BlockSpec((1, D, D), lambda p, b: (p, 0, 0)),
        ],
        out_specs=pl.BlockSpec((1, RA, D), lambda p, b: (p, b, 0)),
        out_shape=jax.ShapeDtypeStruct((NC, N, D), jnp.float32),
    )(h, Wst)


def _sc_agg(seq2n, ed, ews, zeros):
    mesh = plsc.VectorSubcoreMesh(core_axis_name="c", subcore_axis_name="s")

    @functools.partial(
        pl.kernel,
        out_type=jax.ShapeDtypeStruct((NC, NP, D), jnp.float32),
        mesh=mesh,
        scratch_types=[
            pltpu.VMEM((2, C), jnp.int32),
            pltpu.VMEM((2, C), jnp.int32),
            pltpu.VMEM((1, C), jnp.float32),
            pltpu.VMEM((1, C), jnp.float32),
            pltpu.VMEM((C, D), jnp.float32),
            pltpu.VMEM((C, D), jnp.float32),
            pltpu.VMEM_SHARED((NP, D), jnp.float32),
            pltpu.SemaphoreType.DMA,
            pltpu.SemaphoreType.DMA,
            pltpu.SemaphoreType.DMA,
            pltpu.SemaphoreType.DMA,
        ],
    )
    def body(seq_hbm, ed_hbm, ew_hbm, z_hbm, out_hbm,
             ec_a, ec_b, ew_a, ew_b, rows_a, rows_b, agg_sh,
             gs_a, gs_b, is_a, is_b):
        c = lax.axis_index("c")
        s = lax.axis_index("s")
        pltpu.sync_copy(z_hbm, agg_sh.at[pl.ds(s * RPT, RPT)])
        plsc.subcore_barrier()

        def mul_rows(rows, ewbuf):
            def grp(g, carry):
                ewv = ewbuf[0, pl.ds(g * 16, 16)]
                for r in range(16):
                    i = g * 16 + r
                    w = jnp.full((16,), ewv[r], jnp.float32)
                    for k in range(D // 16):
                        sl = pl.ds(k * 16, 16)
                        rows[i, sl] = rows[i, sl] * w
                return carry
            lax.fori_loop(0, C // 16, grp, 0)

        pltpu.sync_copy(ed_hbm.at[c, s, 0], ec_a)
        pltpu.sync_copy(ew_hbm.at[c, s, 0], ew_a)
        pltpu.async_copy(seq_hbm.at[ec_a.at[0]], rows_a, gs_a)
        pltpu.async_copy(ed_hbm.at[c, s, 1], ec_b, is_b)
        pltpu.async_copy(ew_hbm.at[c, s, 1], ew_b, is_b)

        def step(t, carry):
            j = 2 * t
            pltpu.make_async_copy(ed_hbm.at[c, s, 0], ec_b, is_b).wait()
            pltpu.make_async_copy(ew_hbm.at[c, s, 0], ew_b, is_b).wait()
            gd_b = pltpu.async_copy(seq_hbm.at[ec_b.at[0]], rows_b, gs_b)
            pltpu.make_async_copy(seq_hbm.at[ec_a.at[0]], rows_a, gs_a).wait()
            mul_rows(rows_a, ew_a)
            pltpu.sync_copy(rows_a, agg_sh.at[ec_a.at[1]], add=True)

            @pl.when(j + 2 < CH)
            def _():
                pltpu.async_copy(ed_hbm.at[c, s, j + 2], ec_a, is_a)
                pltpu.async_copy(ew_hbm.at[c, s, j + 2], ew_a, is_a)

            @pl.when(j + 2 < CH)
            def _():
                pltpu.make_async_copy(ed_hbm.at[c, s, 0], ec_a, is_a).wait()
                pltpu.make_async_copy(ew_hbm.at[c, s, 0], ew_a, is_a).wait()
                pltpu.async_copy(seq_hbm.at[ec_a.at[0]], rows_a, gs_a)
            gd_b.wait()
            mul_rows(rows_b, ew_b)
            pltpu.sync_copy(rows_b, agg_sh.at[ec_b.at[1]], add=True)

            @pl.when(j + 3 < CH)
            def _():
                pltpu.async_copy(ed_hbm.at[c, s, j + 3], ec_b, is_b)
                pltpu.async_copy(ew_hbm.at[c, s, j + 3], ew_b, is_b)
            return carry

        lax.fori_loop(0, CH // 2, step, 0)
        plsc.subcore_barrier()
        pltpu.sync_copy(agg_sh.at[pl.ds(s * RPT, RPT)],
                        out_hbm.at[c, pl.ds(s * RPT, RPT)])

    return body(seq2n, ed, ews, zeros)


def _film_att(agg, seq, h, ntf, gt, ad, aW1, ab1, aW2, avec):
    R = 1000
    nb = N // R

    def body(agg_ref, seq_ref, h_ref, nt_ref, gt_ref, ad_ref,
             aw1_ref, ab1_ref, aw2_ref, a_ref, o_ref):
        nt = nt_ref[...]

        def z(p):
            g = gt_ref[p, 0:1, :] + nt * (gt_ref[p, 1:2, :] - gt_ref[p, 0:1, :])
            b = ad_ref[p, 0:1, :] + nt * (ad_ref[p, 1:2, :] - ad_ref[p, 0:1, :])
            zz = g * agg_ref[p] + b + seq_ref[p]
            return jnp.where(zz >= 0, zz, a_ref[p] * zz)

        z0 = z(0)
        z1 = z(1)
        aw1 = aw1_ref[...]
        ab1 = ab1_ref[...]
        aw2 = aw2_ref[...]
        w0 = jnp.dot(jnp.tanh(jnp.dot(z0, aw1,
                                      preferred_element_type=jnp.float32)
                              + ab1), aw2,
                     preferred_element_type=jnp.float32)
        w1 = jnp.dot(jnp.tanh(jnp.dot(z1, aw1,
                                      preferred_element_type=jnp.float32)
                              + ab1), aw2,
                     preferred_element_type=jnp.float32)
        m = jnp.maximum(w0, w1)
        e0 = jnp.exp(w0 - m)
        e1 = jnp.exp(w1 - m)
        o_ref[...] = (e0 * z0 + e1 * z1) / (e0 + e1) + h_ref[...]

    full3 = lambda i: (0, 0, 0)
    return pl.pallas_call(
        body,
        grid=(nb,),
        in_specs=[
            pl.BlockSpec((NC, R, D), lambda i: (0, i, 0)),
            pl.BlockSpec((NC, R, D), lambda i: (0, i, 0)),
            pl.BlockSpec((R, D), lambda i: (i, 0)),
            pl.BlockSpec((R, 1), lambda i: (i, 0)),
            pl.BlockSpec((NC, 2, D), full3),
            pl.BlockSpec((NC, 2, D), full3),
            pl.BlockSpec((D, D), lambda i: (0, 0)),
            pl.BlockSpec((1, D), lambda i: (0, 0)),
            pl.BlockSpec((D, 1), lambda i: (0, 0)),
            pl.BlockSpec(memory_space=pltpu.SMEM),
        ],
        out_specs=pl.BlockSpec((R, D), lambda i: (i, 0)),
        out_shape=jax.ShapeDtypeStruct((N, D), jnp.float32),
    )(agg, seq, h, ntf, gt, ad, aW1, ab1, aW2, avec)


def _prep_edges(ei, ew, table_off):
    pad = NS * EPT - E
    src = jnp.concatenate([ei[1] + table_off,
                           jnp.zeros((pad,), jnp.int32)]).reshape(NS, CH, C)
    dst = jnp.concatenate([ei[0],
                           jnp.zeros((pad,), jnp.int32)]).reshape(NS, CH, C)
    eww = jnp.concatenate([ew,
                           jnp.zeros((pad,), jnp.float32)]).reshape(NS, CH, 1, C)
    return jnp.stack([src, dst], axis=2), eww


def kernel(h, edge_index0, edge_index1, edge_weight0, edge_weight1,
           node_type, W0, a0, Wg0, bg0, Wb0, bb0, bias0,
           W1, a1, Wg1, bg1, Wb1, bb1, bias1, attW1, attb1, attW2):
    Wst = jnp.stack([W0, W1])
    seq = _seq_fts(h, Wst)

    ed0, ew0 = _prep_edges(edge_index0, edge_weight0, 0)
    ed1, ew1 = _prep_edges(edge_index1, edge_weight1, N)
    ed = jnp.stack([ed0, ed1])
    ews = jnp.stack([ew0, ew1])
    zeros = jnp.zeros((RPT, D), jnp.float32)

    agg = _sc_agg(seq.reshape(NC * N, D), ed, ews, zeros)

    gt = jnp.stack([Wg0 + bg0[None, :], Wg1 + bg1[None, :]])
    ad = jnp.stack([Wb0 + (bb0 + bias0)[None, :],
                    Wb1 + (bb1 + bias1)[None, :]])
    avec = jnp.stack([a0, a1])
    ntf = node_type.astype(jnp.float32)[:, None]

    return _film_att(agg, seq, h, ntf, gt, ad,
                     attW1, attb1[None, :], attW2, avec)


# speedup vs baseline: 6.1780x; 1.1758x over previous
import functools

import jax
import jax.numpy as jnp
from jax import lax
from jax.experimental import pallas as pl
from jax.experimental.pallas import tpu as pltpu
from jax.experimental.pallas import tpu_sc as plsc

N = 10000
D = 128
E = 320000
NC = 2
NS = 16
C = 112
CH = -(-E // (NS * C))
CH += (-CH) % 3
EPT = CH * C
NP = 10112
RPT = NP // NS


def _seq_fts(h, Wst):
    RA = 2000
    nb = N // RA

    def body(h_ref, w_ref, o_ref):
        o_ref[0] = jnp.dot(h_ref[...], w_ref[0],
                           preferred_element_type=jnp.float32)

    return pl.pallas_call(
        body,
        grid=(NC, nb),
        in_specs=[
            pl.BlockSpec((RA, D), lambda p, b: (b, 0)),
            pl.BlockSpec((1, D, D), lambda p, b: (p, 0, 0)),
        ],
        out_specs=pl.BlockSpec((1, RA, D), lambda p, b: (p, b, 0)),
        out_shape=jax.ShapeDtypeStruct((NC, N, D), jnp.float32),
    )(h, Wst)


def _sc_agg(seq2n, ed, ews, zeros):
    mesh = plsc.VectorSubcoreMesh(core_axis_name="c", subcore_axis_name="s")

    @functools.partial(
        pl.kernel,
        out_type=jax.ShapeDtypeStruct((NC, NP, D), jnp.float32),
        mesh=mesh,
        scratch_types=[
            pltpu.VMEM((2, C), jnp.int32),
            pltpu.VMEM((2, C), jnp.int32),
            pltpu.VMEM((2, C), jnp.int32),
            pltpu.VMEM((C,), jnp.float32),
            pltpu.VMEM((C,), jnp.float32),
            pltpu.VMEM((C,), jnp.float32),
            pltpu.VMEM((C, D), jnp.float32),
            pltpu.VMEM((C, D), jnp.float32),
            pltpu.VMEM((C, D), jnp.float32),
            pltpu.VMEM_SHARED((NP, D), jnp.float32),
            pltpu.SemaphoreType.DMA,
            pltpu.SemaphoreType.DMA,
            pltpu.SemaphoreType.DMA,
            pltpu.SemaphoreType.DMA,
            pltpu.SemaphoreType.DMA,
            pltpu.SemaphoreType.DMA,
            pltpu.SemaphoreType.DMA,
            pltpu.SemaphoreType.DMA,
            pltpu.SemaphoreType.DMA,
        ],
    )
    def body(seq_hbm, ed_hbm, ew_hbm, z_hbm, out_hbm,
             ec0, ec1, ec2, ew0, ew1, ew2, rw0, rw1, rw2, agg_sh,
             gs0, gs1, gs2, is0, is1, is2, ss0, ss1, ss2):
        c = lax.axis_index("c")
        s = lax.axis_index("s")
        ECS, EWS, RWS = (ec0, ec1, ec2), (ew0, ew1, ew2), (rw0, rw1, rw2)
        GS, IS, SS = (gs0, gs1, gs2), (is0, is1, is2), (ss0, ss1, ss2)
        pltpu.sync_copy(z_hbm, agg_sh.at[pl.ds(s * RPT, RPT)])
        plsc.subcore_barrier()

        def mul_rows(rows, ewm):
            def grp(g, carry):
                ewv = ewm[pl.ds(g * 16, 16)]
                for r in range(16):
                    i = g * 16 + r
                    w = jnp.full((16,), ewv[r], jnp.float32)
                    for k in range(D // 16):
                        sl = pl.ds(k * 16, 16)
                        rows[i, sl] = rows[i, sl] * w
                return carry
            lax.fori_loop(0, C // 16, grp, 0)

        pltpu.sync_copy(ed_hbm.at[c, s, 0], ec0)
        pltpu.sync_copy(ew_hbm.at[c, s, 0], ew0)
        pltpu.async_copy(seq_hbm.at[ec0.at[0]], rw0, gs0)
        pltpu.async_copy(ed_hbm.at[c, s, 1], ec1, is1)
        pltpu.async_copy(ew_hbm.at[c, s, 1], ew1, is1)

        def half(j, p):
            n1, n2 = (p + 1) % 3, (p + 2) % 3
            ec_p, ew_p, rw_p = ECS[p], EWS[p], RWS[p]
            ec_1, rw_1 = ECS[n1], RWS[n1]
            ec_2, ew_2 = ECS[n2], EWS[n2]

            @pl.when(j + 1 < CH)
            def _():
                pltpu.make_async_copy(ed_hbm.at[c, s, 0], ec_1, IS[n1]).wait()
                pltpu.make_async_copy(ew_hbm.at[c, s, 0], EWS[n1],
                                      IS[n1]).wait()
                pltpu.async_copy(seq_hbm.at[ec_1.at[0]], rw_1, GS[n1])

            pltpu.make_async_copy(seq_hbm.at[ec_p.at[0]], rw_p, GS[p]).wait()
            mul_rows(rw_p, ew_p)
            pltpu.async_copy(rw_p, agg_sh.at[ec_p.at[1]], SS[p], add=True)

            @pl.when(jnp.logical_and(j >= 1, j + 2 < CH))
            def _():
                pltpu.make_async_copy(RWS[n2], agg_sh.at[ec_2.at[1]],
                                      SS[n2]).wait()

            @pl.when(j + 2 < CH)
            def _():
                pltpu.async_copy(ed_hbm.at[c, s, j + 2], ec_2, IS[n2])
                pltpu.async_copy(ew_hbm.at[c, s, j + 2], ew_2, IS[n2])

        def step(t, carry):
            half(3 * t, 0)
            half(3 * t + 1, 1)
            half(3 * t + 2, 2)
            return carry

        lax.fori_loop(0, CH // 3, step, 0)
        pltpu.make_async_copy(rw0, agg_sh.at[ec0.at[1]], ss0).wait()
        pltpu.make_async_copy(rw1, agg_sh.at[ec1.at[1]], ss1).wait()
        pltpu.make_async_copy(rw2, agg_sh.at[ec2.at[1]], ss2).wait()
        plsc.subcore_barrier()
        pltpu.sync_copy(agg_sh.at[pl.ds(s * RPT, RPT)],
                        out_hbm.at[c, pl.ds(s * RPT, RPT)])

    return body(seq2n, ed, ews, zeros)


def _film_att(agg, seq, h, ntf, gt, ad, aW1, ab1, aW2, avec):
    R = 1000
    nb = N // R

    def body(agg_ref, seq_ref, h_ref, nt_ref, gt_ref, ad_ref,
             aw1_ref, ab1_ref, aw2_ref, a_ref, o_ref):
        nt = nt_ref[...]

        def z(p):
            g = gt_ref[p, 0:1, :] + nt * (gt_ref[p, 1:2, :] - gt_ref[p, 0:1, :])
            b = ad_ref[p, 0:1, :] + nt * (ad_ref[p, 1:2, :] - ad_ref[p, 0:1, :])
            zz = g * agg_ref[p] + b + seq_ref[p]
            return jnp.where(zz >= 0, zz, a_ref[p] * zz)

        z0 = z(0)
        z1 = z(1)
        aw1 = aw1_ref[...]
        ab1 = ab1_ref[...]
        aw2 = aw2_ref[...]
        w0 = jnp.dot(jnp.tanh(jnp.dot(z0, aw1,
                                      preferred_element_type=jnp.float32)
                              + ab1), aw2,
                     preferred_element_type=jnp.float32)
        w1 = jnp.dot(jnp.tanh(jnp.dot(z1, aw1,
                                      preferred_element_type=jnp.float32)
                              + ab1), aw2,
                     preferred_element_type=jnp.float32)
        m = jnp.maximum(w0, w1)
        e0 = jnp.exp(w0 - m)
        e1 = jnp.exp(w1 - m)
        o_ref[...] = (e0 * z0 + e1 * z1) / (e0 + e1) + h_ref[...]

    full3 = lambda i: (0, 0, 0)
    return pl.pallas_call(
        body,
        grid=(nb,),
        in_specs=[
            pl.BlockSpec((NC, R, D), lambda i: (0, i, 0)),
            pl.BlockSpec((NC, R, D), lambda i: (0, i, 0)),
            pl.BlockSpec((R, D), lambda i: (i, 0)),
            pl.BlockSpec((R, 1), lambda i: (i, 0)),
            pl.BlockSpec((NC, 2, D), full3),
            pl.BlockSpec((NC, 2, D), full3),
            pl.BlockSpec((D, D), lambda i: (0, 0)),
            pl.BlockSpec((1, D), lambda i: (0, 0)),
            pl.BlockSpec((D, 1), lambda i: (0, 0)),
            pl.BlockSpec(memory_space=pltpu.SMEM),
        ],
        out_specs=pl.BlockSpec((R, D), lambda i: (i, 0)),
        out_shape=jax.ShapeDtypeStruct((N, D), jnp.float32),
    )(agg, seq, h, ntf, gt, ad, aW1, ab1, aW2, avec)


def _prep_edges(ei, ew, table_off):
    pad = NS * EPT - E
    src = jnp.concatenate([ei[1] + table_off,
                           jnp.zeros((pad,), jnp.int32)]).reshape(NS, CH, C)
    dst = jnp.concatenate([ei[0],
                           jnp.zeros((pad,), jnp.int32)]).reshape(NS, CH, C)
    eww = jnp.concatenate([ew,
                           jnp.zeros((pad,), jnp.float32)]).reshape(NS, CH, C)
    return jnp.stack([src, dst], axis=2), eww


def kernel(h, edge_index0, edge_index1, edge_weight0, edge_weight1,
           node_type, W0, a0, Wg0, bg0, Wb0, bb0, bias0,
           W1, a1, Wg1, bg1, Wb1, bb1, bias1, attW1, attb1, attW2):
    Wst = jnp.stack([W0, W1])
    seq = _seq_fts(h, Wst)

    ed0, ew0 = _prep_edges(edge_index0, edge_weight0, 0)
    ed1, ew1 = _prep_edges(edge_index1, edge_weight1, N)
    ed = jnp.stack([ed0, ed1])
    ews = jnp.stack([ew0, ew1])
    zeros = jnp.zeros((RPT, D), jnp.float32)

    agg = _sc_agg(seq.reshape(NC * N, D), ed, ews, zeros)

    gt = jnp.stack([Wg0 + bg0[None, :], Wg1 + bg1[None, :]])
    ad = jnp.stack([Wb0 + (bb0 + bias0)[None, :],
                    Wb1 + (bb1 + bias1)[None, :]])
    avec = jnp.stack([a0, a1])
    ntf = node_type.astype(jnp.float32)[:, None]

    return _film_att(agg, seq, h, ntf, gt, ad,
                     attW1, attb1[None, :], attW2, avec)


# speedup vs baseline: 6.2506x; 1.0118x over previous
import functools

import jax
import jax.numpy as jnp
from jax import lax
from jax.experimental import pallas as pl
from jax.experimental.pallas import tpu as pltpu
from jax.experimental.pallas import tpu_sc as plsc

N = 10000
D = 128
E = 320000
NC = 2
NS = 16
C = 112
CH = -(-E // (NS * C))
CH += (-CH) % 3
EPT = CH * C
NP = 10112
RPT = NP // NS


def _seq_fts(h, Wst):
    RA = 5000
    nb = N // RA

    def body(h_ref, w_ref, o_ref):
        o_ref[0] = jnp.dot(h_ref[...], w_ref[0],
                           preferred_element_type=jnp.float32)

    return pl.pallas_call(
        body,
        grid=(NC, nb),
        in_specs=[
            pl.BlockSpec((RA, D), lambda p, b: (b, 0)),
            pl.BlockSpec((1, D, D), lambda p, b: (p, 0, 0)),
        ],
        out_specs=pl.BlockSpec((1, RA, D), lambda p, b: (p, b, 0)),
        out_shape=jax.ShapeDtypeStruct((NC, N, D), jnp.float32),
    )(h, Wst)


def _sc_agg(seq2n, ed, ews, zeros):
    mesh = plsc.VectorSubcoreMesh(core_axis_name="c", subcore_axis_name="s")

    @functools.partial(
        pl.kernel,
        out_type=jax.ShapeDtypeStruct((NC, NP, D), jnp.float32),
        mesh=mesh,
        scratch_types=[
            pltpu.VMEM((2, C), jnp.int32),
            pltpu.VMEM((2, C), jnp.int32),
            pltpu.VMEM((2, C), jnp.int32),
            pltpu.VMEM((C,), jnp.float32),
            pltpu.VMEM((C,), jnp.float32),
            pltpu.VMEM((C,), jnp.float32),
            pltpu.VMEM((C, D), jnp.float32),
            pltpu.VMEM((C, D), jnp.float32),
            pltpu.VMEM((C, D), jnp.float32),
            pltpu.VMEM_SHARED((NP, D), jnp.float32),
            pltpu.SemaphoreType.DMA,
            pltpu.SemaphoreType.DMA,
            pltpu.SemaphoreType.DMA,
            pltpu.SemaphoreType.DMA,
            pltpu.SemaphoreType.DMA,
            pltpu.SemaphoreType.DMA,
            pltpu.SemaphoreType.DMA,
            pltpu.SemaphoreType.DMA,
            pltpu.SemaphoreType.DMA,
        ],
    )
    def body(seq_hbm, ed_hbm, ew_hbm, z_hbm, out_hbm,
             ec0, ec1, ec2, ew0, ew1, ew2, rw0, rw1, rw2, agg_sh,
             gs0, gs1, gs2, is0, is1, is2, ss0, ss1, ss2):
        c = lax.axis_index("c")
        s = lax.axis_index("s")
        ECS, EWS, RWS = (ec0, ec1, ec2), (ew0, ew1, ew2), (rw0, rw1, rw2)
        GS, IS, SS = (gs0, gs1, gs2), (is0, is1, is2), (ss0, ss1, ss2)
        pltpu.sync_copy(z_hbm, agg_sh.at[pl.ds(s * RPT, RPT)])
        plsc.subcore_barrier()

        def mul_rows(rows, ewm):
            def grp(g, carry):
                ewv = ewm[pl.ds(g * 16, 16)]
                for r in range(16):
                    i = g * 16 + r
                    w = jnp.full((16,), ewv[r], jnp.float32)
                    for k in range(D // 16):
                        sl = pl.ds(k * 16, 16)
                        rows[i, sl] = rows[i, sl] * w
                return carry
            lax.fori_loop(0, C // 16, grp, 0)

        pltpu.sync_copy(ed_hbm.at[c, s, 0], ec0)
        pltpu.sync_copy(ew_hbm.at[c, s, 0], ew0)
        pltpu.async_copy(seq_hbm.at[ec0.at[0]], rw0, gs0)
        pltpu.async_copy(ed_hbm.at[c, s, 1], ec1, is1)
        pltpu.async_copy(ew_hbm.at[c, s, 1], ew1, is1)

        def half(j, p):
            n1, n2 = (p + 1) % 3, (p + 2) % 3
            ec_p, ew_p, rw_p = ECS[p], EWS[p], RWS[p]
            ec_1, rw_1 = ECS[n1], RWS[n1]
            ec_2, ew_2 = ECS[n2], EWS[n2]

            @pl.when(j + 1 < CH)
            def _():
                pltpu.make_async_copy(ed_hbm.at[c, s, 0], ec_1, IS[n1]).wait()
                pltpu.make_async_copy(ew_hbm.at[c, s, 0], EWS[n1],
                                      IS[n1]).wait()
                pltpu.async_copy(seq_hbm.at[ec_1.at[0]], rw_1, GS[n1])

            pltpu.make_async_copy(seq_hbm.at[ec_p.at[0]], rw_p, GS[p]).wait()
            mul_rows(rw_p, ew_p)
            pltpu.async_copy(rw_p, agg_sh.at[ec_p.at[1]], SS[p], add=True)

            @pl.when(jnp.logical_and(j >= 1, j + 2 < CH))
            def _():
                pltpu.make_async_copy(RWS[n2], agg_sh.at[ec_2.at[1]],
                                      SS[n2]).wait()

            @pl.when(j + 2 < CH)
            def _():
                pltpu.async_copy(ed_hbm.at[c, s, j + 2], ec_2, IS[n2])
                pltpu.async_copy(ew_hbm.at[c, s, j + 2], ew_2, IS[n2])

        def step(t, carry):
            half(3 * t, 0)
            half(3 * t + 1, 1)
            half(3 * t + 2, 2)
            return carry

        lax.fori_loop(0, CH // 3, step, 0)
        pltpu.make_async_copy(rw0, agg_sh.at[ec0.at[1]], ss0).wait()
        pltpu.make_async_copy(rw1, agg_sh.at[ec1.at[1]], ss1).wait()
        pltpu.make_async_copy(rw2, agg_sh.at[ec2.at[1]], ss2).wait()
        plsc.subcore_barrier()
        pltpu.sync_copy(agg_sh.at[pl.ds(s * RPT, RPT)],
                        out_hbm.at[c, pl.ds(s * RPT, RPT)])

    return body(seq2n, ed, ews, zeros)


def _film_att(agg, seq, h, ntf, gt, ad, aW1, ab1, aW2, avec):
    R = 2000
    nb = N // R

    def body(agg_ref, seq_ref, h_ref, nt_ref, gt_ref, ad_ref,
             aw1_ref, ab1_ref, aw2_ref, a_ref, o_ref):
        nt = nt_ref[...]

        def z(p):
            g = gt_ref[p, 0:1, :] + nt * (gt_ref[p, 1:2, :] - gt_ref[p, 0:1, :])
            b = ad_ref[p, 0:1, :] + nt * (ad_ref[p, 1:2, :] - ad_ref[p, 0:1, :])
            zz = g * agg_ref[p] + b + seq_ref[p]
            return jnp.where(zz >= 0, zz, a_ref[p] * zz)

        z0 = z(0)
        z1 = z(1)
        aw1 = aw1_ref[...]
        ab1 = ab1_ref[...]
        aw2 = aw2_ref[...]
        w0 = jnp.dot(jnp.tanh(jnp.dot(z0, aw1,
                                      preferred_element_type=jnp.float32)
                              + ab1), aw2,
                     preferred_element_type=jnp.float32)
        w1 = jnp.dot(jnp.tanh(jnp.dot(z1, aw1,
                                      preferred_element_type=jnp.float32)
                              + ab1), aw2,
                     preferred_element_type=jnp.float32)
        m = jnp.maximum(w0, w1)
        e0 = jnp.exp(w0 - m)
        e1 = jnp.exp(w1 - m)
        o_ref[...] = (e0 * z0 + e1 * z1) / (e0 + e1) + h_ref[...]

    full3 = lambda i: (0, 0, 0)
    return pl.pallas_call(
        body,
        grid=(nb,),
        in_specs=[
            pl.BlockSpec((NC, R, D), lambda i: (0, i, 0)),
            pl.BlockSpec((NC, R, D), lambda i: (0, i, 0)),
            pl.BlockSpec((R, D), lambda i: (i, 0)),
            pl.BlockSpec((R, 1), lambda i: (i, 0)),
            pl.BlockSpec((NC, 2, D), full3),
            pl.BlockSpec((NC, 2, D), full3),
            pl.BlockSpec((D, D), lambda i: (0, 0)),
            pl.BlockSpec((1, D), lambda i: (0, 0)),
            pl.BlockSpec((D, 1), lambda i: (0, 0)),
            pl.BlockSpec(memory_space=pltpu.SMEM),
        ],
        out_specs=pl.BlockSpec((R, D), lambda i: (i, 0)),
        out_shape=jax.ShapeDtypeStruct((N, D), jnp.float32),
    )(agg, seq, h, ntf, gt, ad, aW1, ab1, aW2, avec)


def _prep_edges(ei, ew, table_off):
    pad = NS * EPT - E
    src = jnp.concatenate([ei[1] + table_off,
                           jnp.zeros((pad,), jnp.int32)]).reshape(NS, CH, C)
    dst = jnp.concatenate([ei[0],
                           jnp.zeros((pad,), jnp.int32)]).reshape(NS, CH, C)
    eww = jnp.concatenate([ew,
                           jnp.zeros((pad,), jnp.float32)]).reshape(NS, CH, C)
    return jnp.stack([src, dst], axis=2), eww


def kernel(h, edge_index0, edge_index1, edge_weight0, edge_weight1,
           node_type, W0, a0, Wg0, bg0, Wb0, bb0, bias0,
           W1, a1, Wg1, bg1, Wb1, bb1, bias1, attW1, attb1, attW2):
    Wst = jnp.stack([W0, W1])
    seq = _seq_fts(h, Wst)

    ed0, ew0 = _prep_edges(edge_index0, edge_weight0, 0)
    ed1, ew1 = _prep_edges(edge_index1, edge_weight1, N)
    ed = jnp.stack([ed0, ed1])
    ews = jnp.stack([ew0, ew1])
    zeros = jnp.zeros((RPT, D), jnp.float32)

    agg = _sc_agg(seq.reshape(NC * N, D), ed, ews, zeros)

    gt = jnp.stack([Wg0 + bg0[None, :], Wg1 + bg1[None, :]])
    ad = jnp.stack([Wb0 + (bb0 + bias0)[None, :],
                    Wb1 + (bb1 + bias1)[None, :]])
    avec = jnp.stack([a0, a1])
    ntf = node_type.astype(jnp.float32)[:, None]

    return _film_att(agg, seq, h, ntf, gt, ad,
                     attW1, attb1[None, :], attW2, avec)
